# Initial kernel scaffold; baseline (speedup 1.0000x reference)
import jax
import jax.numpy as jnp
from jax.experimental import pallas as pl


def kernel(xyz1, xyz2):
    raise NotImplementedError("write your pallas kernel here")


# speedup vs baseline: 1.0532x; 1.0532x over previous
import jax
import jax.numpy as jnp
from jax.experimental import pallas as pl
from jax.experimental.pallas import tpu as pltpu

_B, _N, _M = 2, 4096, 3
_CHUNK = 512


def _chamfer_body(x1_ref, x2t_ref, out_ref):
    B, N, _ = x1_ref.shape
    M = x2t_ref.shape[2]
    n_chunks = N // _CHUNK

    total = jnp.float32(0.0)
    for b in range(B):
        G = x2t_ref[b]
        r2 = jnp.sum(G * G, axis=0, keepdims=True)

        def chunk_step(i, carry):
            sum1, min2 = carry
            q = x1_ref[b, pl.ds(i * _CHUNK, _CHUNK), :]
            q2 = jnp.sum(q * q, axis=1, keepdims=True)
            xy = jax.lax.dot_general(
                q, G, (((1,), (0,)), ((), ())),
                preferred_element_type=jnp.float32,
            )
            d = jnp.maximum(q2 + r2 - 2.0 * xy, 0.0)
            sum1 = sum1 + jnp.sum(jnp.min(d, axis=1))
            min2 = jnp.minimum(min2, jnp.min(d, axis=0))
            return sum1, min2

        sum1, min2 = jax.lax.fori_loop(
            0, n_chunks, chunk_step,
            (jnp.float32(0.0), jnp.full((M,), jnp.inf, jnp.float32)),
        )
        total = total + sum1 / (B * N) + jnp.sum(min2) / (B * M)

    out_ref[0, 0] = total


def kernel(xyz1, xyz2):
    x2t = jnp.transpose(xyz2, (0, 2, 1))
    out = pl.pallas_call(
        _chamfer_body,
        out_shape=jax.ShapeDtypeStruct((1, 1), jnp.float32),
        out_specs=pl.BlockSpec(memory_space=pltpu.SMEM),
    )(xyz1, x2t)
    return out[0, 0]


# speedup vs baseline: 1.3688x; 1.2997x over previous
import jax
import jax.numpy as jnp
from jax.experimental import pallas as pl
from jax.experimental.pallas import tpu as pltpu

_B, _N, _M = 2, 4096, 3
_CHUNK = 512


def _chamfer_body(x1_ref, x2t_ref, out_ref):
    B, N, _ = x1_ref.shape
    M = x2t_ref.shape[2]
    n_chunks = N // _CHUNK

    total = jnp.float32(0.0)
    for b in range(B):
        G = x2t_ref[b]
        r2 = jnp.sum(G * G, axis=0, keepdims=True)
        ones_m = jnp.ones((1, M), jnp.float32)
        Ga = jnp.concatenate([G, ones_m, r2], axis=0)

        def chunk_step(i, carry):
            sum1, min2 = carry
            q = x1_ref[b, pl.ds(i * _CHUNK, _CHUNK), :]
            q2 = jnp.sum(q * q, axis=1, keepdims=True)
            qa = jnp.concatenate(
                [-2.0 * q, q2, jnp.ones((_CHUNK, 1), jnp.float32)], axis=1
            )
            d = jax.lax.dot_general(
                qa, Ga, (((1,), (0,)), ((), ())),
                preferred_element_type=jnp.float32,
            )
            sum1 = sum1 + jnp.sum(jnp.maximum(jnp.min(d, axis=1), 0.0))
            min2 = jnp.minimum(min2, jnp.min(d, axis=0))
            return sum1, min2

        sum1, min2 = jax.lax.fori_loop(
            0, n_chunks, chunk_step,
            (jnp.float32(0.0), jnp.full((M,), jnp.inf, jnp.float32)),
        )
        total = total + sum1 / (B * N) + jnp.sum(jnp.maximum(min2, 0.0)) / (B * M)

    out_ref[0, 0] = total


def kernel(xyz1, xyz2):
    x2t = jnp.transpose(xyz2, (0, 2, 1))
    out = pl.pallas_call(
        _chamfer_body,
        out_shape=jax.ShapeDtypeStruct((1, 1), jnp.float32),
        out_specs=pl.BlockSpec(memory_space=pltpu.SMEM),
    )(xyz1, x2t)
    return out[0, 0]


# speedup vs baseline: 1.5228x; 1.1125x over previous
import jax
import jax.numpy as jnp
from jax.experimental import pallas as pl
from jax.experimental.pallas import tpu as pltpu

_B, _N, _M = 2, 4096, 3
_CHUNK = 1024


def _chamfer_body(x1_ref, x2t_ref, out_ref):
    B, N, _ = x1_ref.shape
    M = x2t_ref.shape[2]
    n_chunks = N // _CHUNK

    total = jnp.float32(0.0)
    for b in range(B):
        G = x2t_ref[b]
        r2 = jnp.sum(G * G, axis=0, keepdims=True)
        ones_m = jnp.ones((1, M), jnp.float32)
        Ga = jnp.concatenate([G, ones_m, r2], axis=0)

        def chunk_step(i, carry):
            sum1, min2 = carry
            q = x1_ref[b, pl.ds(i * _CHUNK, _CHUNK), :]
            q2 = jnp.sum(q * q, axis=1, keepdims=True)
            qa = jnp.concatenate(
                [-2.0 * q, q2, jnp.ones((_CHUNK, 1), jnp.float32)], axis=1
            )
            d = jax.lax.dot_general(
                qa, Ga, (((1,), (0,)), ((), ())),
                preferred_element_type=jnp.float32,
            )
            sum1 = sum1 + jnp.sum(jnp.maximum(jnp.min(d, axis=1), 0.0))
            min2 = jnp.minimum(min2, jnp.min(d, axis=0))
            return sum1, min2

        sum1, min2 = jax.lax.fori_loop(
            0, n_chunks, chunk_step,
            (jnp.float32(0.0), jnp.full((M,), jnp.inf, jnp.float32)),
        )
        total = total + sum1 / (B * N) + jnp.sum(jnp.maximum(min2, 0.0)) / (B * M)

    out_ref[0, 0] = total


def kernel(xyz1, xyz2):
    x2t = jnp.transpose(xyz2, (0, 2, 1))
    out = pl.pallas_call(
        _chamfer_body,
        out_shape=jax.ShapeDtypeStruct((1, 1), jnp.float32),
        out_specs=pl.BlockSpec(memory_space=pltpu.SMEM),
    )(xyz1, x2t)
    return out[0, 0]


# speedup vs baseline: 1.6096x; 1.0570x over previous
import jax
import jax.numpy as jnp
from jax.experimental import pallas as pl
from jax.experimental.pallas import tpu as pltpu

_B, _N, _M = 2, 4096, 3
_CHUNK = 2048


def _chamfer_body(x1_ref, x2t_ref, out_ref):
    B, N, _ = x1_ref.shape
    M = x2t_ref.shape[2]
    n_chunks = N // _CHUNK

    total = jnp.float32(0.0)
    for b in range(B):
        G = x2t_ref[b]
        r2 = jnp.sum(G * G, axis=0, keepdims=True)
        ones_m = jnp.ones((1, M), jnp.float32)
        Ga = jnp.concatenate([G, ones_m, r2], axis=0)

        def chunk_step(i, carry):
            sum1, min2 = carry
            q = x1_ref[b, pl.ds(i * _CHUNK, _CHUNK), :]
            q2 = jnp.sum(q * q, axis=1, keepdims=True)
            qa = jnp.concatenate(
                [-2.0 * q, q2, jnp.ones((_CHUNK, 1), jnp.float32)], axis=1
            )
            d = jax.lax.dot_general(
                qa, Ga, (((1,), (0,)), ((), ())),
                preferred_element_type=jnp.float32,
            )
            sum1 = sum1 + jnp.sum(jnp.maximum(jnp.min(d, axis=1), 0.0))
            min2 = jnp.minimum(min2, jnp.min(d, axis=0))
            return sum1, min2

        sum1, min2 = jax.lax.fori_loop(
            0, n_chunks, chunk_step,
            (jnp.float32(0.0), jnp.full((M,), jnp.inf, jnp.float32)),
        )
        total = total + sum1 / (B * N) + jnp.sum(jnp.maximum(min2, 0.0)) / (B * M)

    out_ref[0, 0] = total


def kernel(xyz1, xyz2):
    x2t = jnp.transpose(xyz2, (0, 2, 1))
    out = pl.pallas_call(
        _chamfer_body,
        out_shape=jax.ShapeDtypeStruct((1, 1), jnp.float32),
        out_specs=pl.BlockSpec(memory_space=pltpu.SMEM),
    )(xyz1, x2t)
    return out[0, 0]
